# baseline (speedup 1.0000x reference)
import functools

import jax
import jax.numpy as jnp
from jax import lax
from jax.experimental import pallas as pl
from jax.experimental.pallas import tpu as pltpu
from jax.experimental.pallas import tpu_sc as plsc

B, C, H, W = 4, 96, 384, 384
Hp, Wp = H // 2, W // 2
P = B * C
NW = 32
PPW = P // NW
R = 48
NCH = Hp // R
TOT = PPW * NCH
VPR = Wp // 16
OUT_CH = 2 * R * W

_mesh = plsc.VectorSubcoreMesh(core_axis_name="c", subcore_axis_name="s")


@functools.partial(
    pl.kernel,
    mesh=_mesh,
    out_type=jax.ShapeDtypeStruct((P, H, W), jnp.float32),
    scratch_types=[
        pltpu.VMEM((R, Wp), jnp.float32),
        pltpu.VMEM((R, Wp), jnp.float32),
        pltpu.VMEM((R, Wp), jnp.int32),
        pltpu.VMEM((R, Wp), jnp.int32),
        pltpu.VMEM((2 * R, W), jnp.float32),
        pltpu.VMEM((2 * R, W), jnp.float32),
        pltpu.SemaphoreType.DMA,
        pltpu.SemaphoreType.DMA,
        pltpu.SemaphoreType.DMA,
        pltpu.SemaphoreType.DMA,
    ],
    compiler_params=pltpu.CompilerParams(needs_layout_passes=False),
)
def _unpool(x_hbm, idx_hbm, out_hbm, x0, x1, i0, i1, o0, o1, si0, si1, so0, so1):
    xs, idxs, outs = [x0, x1], [i0, i1], [o0, o1]
    sis, sos = [si0, si1], [so0, so1]
    wid = lax.axis_index("s") * 2 + lax.axis_index("c")
    base_plane = wid * PPW

    def refs_of(g):
        plane = base_plane + (g >> 2)
        r0 = (g & 3) * R
        return plane, r0

    def issue_in(g, b):
        plane, r0 = refs_of(g)
        pltpu.async_copy(x_hbm.at[plane, pl.ds(r0, R), :], xs[b], sis[b])
        pltpu.async_copy(idx_hbm.at[plane, pl.ds(r0, R), :], idxs[b], sis[b])

    def wait_in(g, b):
        plane, r0 = refs_of(g)
        pltpu.make_async_copy(x_hbm.at[plane, pl.ds(r0, R), :], xs[b], sis[b]).wait()
        pltpu.make_async_copy(idx_hbm.at[plane, pl.ds(r0, R), :], idxs[b], sis[b]).wait()

    def out_ref_of(g):
        plane, r0 = refs_of(g)
        return out_hbm.at[plane, pl.ds(2 * r0, 2 * R), :]

    def wait_out(g, b):
        pltpu.make_async_copy(outs[b], out_ref_of(g), sos[b]).wait()

    lanes = lax.iota(jnp.int32, 16)

    def chunk(g, b, first=False, issue_next=True):
        if issue_next:
            issue_in(g + 1, 1 - b)
        if not first:
            wait_out(g, b)
        wait_in(g, b)
        out_v = outs[b]
        x_v, idx_v = xs[b], idxs[b]
        _, r0 = refs_of(g)
        zero16 = jnp.zeros((16,), jnp.float32)

        @plsc.parallel_loop(0, R, unroll=3)
        def _windows(il):
            rbase = (r0 + il) * (2 * W)
            hv0 = jnp.full((16,), 2 * il, jnp.int32)
            hv1 = hv0 + 1
            for cv in range(VPR):
                wv0 = (cv * 32) + 2 * lanes
                wv1 = wv0 + 1
                iv = idx_v[il, pl.ds(cv * 16, 16)]
                xv = x_v[il, pl.ds(cv * 16, 16)]
                rel = iv - rbase
                dr1 = rel >= W
                dc1 = (rel & 1) == 1
                plsc.store_scatter(out_v, [hv0, wv0], jnp.where(dr1 | dc1, zero16, xv))
                plsc.store_scatter(out_v, [hv0, wv1], jnp.where(dr1 | (~dc1), zero16, xv))
                plsc.store_scatter(out_v, [hv1, wv0], jnp.where((~dr1) | dc1, zero16, xv))
                plsc.store_scatter(out_v, [hv1, wv1], jnp.where((~dr1) | (~dc1), zero16, xv))

        pltpu.async_copy(out_v, out_ref_of(g), sos[b])

    issue_in(0, 0)
    chunk(0, 0, first=True)
    chunk(1, 1, first=True)

    def pair(g2, _):
        g = g2 * 2
        chunk(g, 0)
        chunk(g + 1, 1)
        return ()

    lax.fori_loop(1, TOT // 2 - 1, pair, ())

    chunk(TOT - 2, 0)
    chunk(TOT - 1, 1, issue_next=False)

    wait_out(TOT - 2, 0)
    wait_out(TOT - 1, 1)


def kernel(x, indices):
    out = _unpool(x.reshape(P, Hp, Wp), indices.reshape(P, Hp, Wp))
    return out.reshape(B, C, H, W)


# speedup vs baseline: 1.1032x; 1.1032x over previous
import functools

import jax
import jax.numpy as jnp
from jax import lax
from jax.experimental import pallas as pl
from jax.experimental.pallas import tpu as pltpu
from jax.experimental.pallas import tpu_sc as plsc

B, C, H, W = 4, 96, 384, 384
Hp, Wp = H // 2, W // 2
P = B * C
NW = 32
PPW = P // NW
R = 48
NCH = Hp // R
TOT = PPW * NCH
VPR = Wp // 16
OUT_CH = 2 * R * W

_mesh = plsc.VectorSubcoreMesh(core_axis_name="c", subcore_axis_name="s")


@functools.partial(
    pl.kernel,
    mesh=_mesh,
    out_type=jax.ShapeDtypeStruct((P, H, W), jnp.float32),
    scratch_types=[
        pltpu.VMEM((R, Wp), jnp.float32),
        pltpu.VMEM((R, Wp), jnp.float32),
        pltpu.VMEM((R, Wp), jnp.int32),
        pltpu.VMEM((R, Wp), jnp.int32),
        pltpu.VMEM((2 * R, W), jnp.float32),
        pltpu.VMEM((2 * R, W), jnp.float32),
        pltpu.SemaphoreType.DMA,
        pltpu.SemaphoreType.DMA,
        pltpu.SemaphoreType.DMA,
        pltpu.SemaphoreType.DMA,
    ],
    compiler_params=pltpu.CompilerParams(needs_layout_passes=False),
)
def _unpool(x_hbm, idx_hbm, out_hbm, x0, x1, i0, i1, o0, o1, si0, si1, so0, so1):
    xs, idxs, outs = [x0, x1], [i0, i1], [o0, o1]
    sis, sos = [si0, si1], [so0, so1]
    wid = lax.axis_index("s") * 2 + lax.axis_index("c")
    base_plane = wid * PPW

    def refs_of(g):
        plane = base_plane + (g >> 2)
        r0 = (g & 3) * R
        return plane, r0

    def issue_in(g, b):
        plane, r0 = refs_of(g)
        pltpu.async_copy(x_hbm.at[plane, pl.ds(r0, R), :], xs[b], sis[b])
        pltpu.async_copy(idx_hbm.at[plane, pl.ds(r0, R), :], idxs[b], sis[b])

    def wait_in(g, b):
        plane, r0 = refs_of(g)
        pltpu.make_async_copy(x_hbm.at[plane, pl.ds(r0, R), :], xs[b], sis[b]).wait()
        pltpu.make_async_copy(idx_hbm.at[plane, pl.ds(r0, R), :], idxs[b], sis[b]).wait()

    def out_ref_of(g):
        plane, r0 = refs_of(g)
        return out_hbm.at[plane, pl.ds(2 * r0, 2 * R), :]

    def wait_out(g, b):
        pltpu.make_async_copy(outs[b], out_ref_of(g), sos[b]).wait()

    lanes = lax.iota(jnp.int32, 16)

    def chunk(g, b, first=False, issue_next=True):
        if issue_next:
            issue_in(g + 1, 1 - b)
        if not first:
            wait_out(g, b)
        wait_in(g, b)
        out_v = outs[b]
        x_v, idx_v = xs[b], idxs[b]
        _, r0 = refs_of(g)
        zero16 = jnp.zeros((16,), jnp.float32)

        @plsc.parallel_loop(0, R, unroll=1)
        def _windows(il):
            rbase = (r0 + il) * (2 * W)
            hv0 = jnp.full((16,), 2 * il, jnp.int32)
            hv1 = hv0 + 1
            for cv in range(VPR):
                wv0 = (cv * 32) + 2 * lanes
                wv1 = wv0 + 1
                iv = idx_v[il, pl.ds(cv * 16, 16)]
                xv = x_v[il, pl.ds(cv * 16, 16)]
                rel = iv - rbase
                dr1 = rel >= W
                dc1 = (rel & 1) == 1
                plsc.store_scatter(out_v, [hv0, wv0], jnp.where(dr1 | dc1, zero16, xv))
                plsc.store_scatter(out_v, [hv0, wv1], jnp.where(dr1 | (~dc1), zero16, xv))
                plsc.store_scatter(out_v, [hv1, wv0], jnp.where((~dr1) | dc1, zero16, xv))
                plsc.store_scatter(out_v, [hv1, wv1], jnp.where((~dr1) | (~dc1), zero16, xv))

        pltpu.async_copy(out_v, out_ref_of(g), sos[b])

    issue_in(0, 0)
    chunk(0, 0, first=True)
    chunk(1, 1, first=True)

    def pair(g2, _):
        g = g2 * 2
        chunk(g, 0)
        chunk(g + 1, 1)
        return ()

    lax.fori_loop(1, TOT // 2 - 1, pair, ())

    chunk(TOT - 2, 0)
    chunk(TOT - 1, 1, issue_next=False)

    wait_out(TOT - 2, 0)
    wait_out(TOT - 1, 1)


def kernel(x, indices):
    out = _unpool(x.reshape(P, Hp, Wp), indices.reshape(P, Hp, Wp))
    return out.reshape(B, C, H, W)


# speedup vs baseline: 1.1055x; 1.0021x over previous
import functools

import jax
import jax.numpy as jnp
from jax import lax
from jax.experimental import pallas as pl
from jax.experimental.pallas import tpu as pltpu
from jax.experimental.pallas import tpu_sc as plsc

B, C, H, W = 4, 96, 384, 384
Hp, Wp = H // 2, W // 2
P = B * C
NW = 32
PPW = P // NW
R = 48
NCH = Hp // R
TOT = PPW * NCH
VPR = Wp // 16

_mesh = plsc.VectorSubcoreMesh(core_axis_name="c", subcore_axis_name="s")


@functools.partial(
    pl.kernel,
    mesh=_mesh,
    out_type=jax.ShapeDtypeStruct((P, H, W), jnp.float32),
    scratch_types=[
        pltpu.VMEM((R, Wp), jnp.float32),
        pltpu.VMEM((R, Wp), jnp.float32),
        pltpu.VMEM((R, Wp), jnp.int32),
        pltpu.VMEM((R, Wp), jnp.int32),
        pltpu.VMEM((2 * R, W), jnp.float32),
        pltpu.VMEM((2 * R, W), jnp.float32),
        pltpu.SemaphoreType.DMA,
        pltpu.SemaphoreType.DMA,
        pltpu.SemaphoreType.DMA,
        pltpu.SemaphoreType.DMA,
    ],
    compiler_params=pltpu.CompilerParams(needs_layout_passes=False),
)
def _unpool(x_hbm, idx_hbm, out_hbm, x0, x1, i0, i1, o0, o1, si0, si1, so0, so1):
    xs, idxs, outs = [x0, x1], [i0, i1], [o0, o1]
    sis, sos = [si0, si1], [so0, so1]
    wid = lax.axis_index("s") * 2 + lax.axis_index("c")
    base_plane = wid * PPW

    def refs_of(g):
        plane = base_plane + (g >> 2)
        r0 = (g & 3) * R
        return plane, r0

    def issue_in(g, b):
        plane, r0 = refs_of(g)
        pltpu.async_copy(x_hbm.at[plane, pl.ds(r0, R), :], xs[b], sis[b])
        pltpu.async_copy(idx_hbm.at[plane, pl.ds(r0, R), :], idxs[b], sis[b])

    def wait_in(g, b):
        plane, r0 = refs_of(g)
        pltpu.make_async_copy(x_hbm.at[plane, pl.ds(r0, R), :], xs[b], sis[b]).wait()
        pltpu.make_async_copy(idx_hbm.at[plane, pl.ds(r0, R), :], idxs[b], sis[b]).wait()

    def out_ref_of(g):
        plane, r0 = refs_of(g)
        return out_hbm.at[plane, pl.ds(2 * r0, 2 * R), :]

    def wait_out(g, b):
        pltpu.make_async_copy(outs[b], out_ref_of(g), sos[b]).wait()

    lanes = lax.iota(jnp.int32, 16)

    def chunk(g, b, first=False, issue_next=True):
        if issue_next:
            issue_in(g + 1, 1 - b)
        if not first:
            wait_out(g, b)
        wait_in(g, b)
        out_v = outs[b]
        x_v, idx_v = xs[b], idxs[b]
        _, r0 = refs_of(g)
        zero16 = jnp.zeros((16,), jnp.float32)

        @plsc.parallel_loop(0, R, unroll=1)
        def _windows(il):
            rbase = (r0 + il) * (2 * W)
            hv0 = jnp.full((16,), 2 * il, jnp.int32)
            hv1 = hv0 + 1
            for cv in range(VPR):
                wv0 = (cv * 32) + 2 * lanes
                wv1 = wv0 + 1
                iv = idx_v[il, pl.ds(cv * 16, 16)]
                xv = x_v[il, pl.ds(cv * 16, 16)]
                rel = iv - rbase
                dr1 = rel >= W
                dc1 = (rel & 1) == 1
                plsc.store_scatter(out_v, [hv0, wv0], jnp.where(dr1 | dc1, zero16, xv))
                plsc.store_scatter(out_v, [hv0, wv1], jnp.where(dr1 | (~dc1), zero16, xv))
                plsc.store_scatter(out_v, [hv1, wv0], jnp.where((~dr1) | dc1, zero16, xv))
                plsc.store_scatter(out_v, [hv1, wv1], jnp.where((~dr1) | (~dc1), zero16, xv))

        pltpu.async_copy(out_v, out_ref_of(g), sos[b])

    issue_in(0, 0)
    chunk(0, 0, first=True)
    chunk(1, 1, first=True)

    def pair(g2, _):
        g = g2 * 2
        chunk(g, 0)
        chunk(g + 1, 1)
        return ()

    lax.fori_loop(1, TOT // 2 - 1, pair, ())

    chunk(TOT - 2, 0)
    chunk(TOT - 1, 1, issue_next=False)

    wait_out(TOT - 2, 0)
    wait_out(TOT - 1, 1)


def kernel(x, indices):
    out = _unpool(x.reshape(P, Hp, Wp), indices.reshape(P, Hp, Wp))
    return out.reshape(B, C, H, W)
